# baseline (device time: 165482 ns/iter reference)
import jax
import jax.numpy as jnp
from jax import lax
from jax.experimental import pallas as pl
from jax.experimental.pallas import tpu as pltpu

N_DEV = 4


def kernel(t, W):
    m_per, k = t.shape
    _, n = W.shape
    m_chunk = m_per // N_DEV

    def body(t_ref, w_ref, out_ref, rs_ref, ag_ref,
             rs_send_sems, rs_recv_sems, ag_send_sems, ag_recv_sems):
        my = lax.axis_index("i")
        left = (my - 1) % N_DEV
        right = (my + 1) % N_DEV

        barrier_sem = pltpu.get_barrier_semaphore()
        for nbr in (left, right):
            pl.semaphore_signal(
                barrier_sem, inc=1,
                device_id=(nbr,), device_id_type=pl.DeviceIdType.MESH,
            )
        pl.semaphore_wait(barrier_sem, 2)

        def t_chunk(c):
            return t_ref[pl.ds(c * m_chunk, m_chunk), :].astype(jnp.bfloat16)

        rs_ref[0, :, :] = t_chunk((my - 1) % N_DEV)
        for h in range(N_DEV - 1):
            s, r = h % 2, (h + 1) % 2
            rdma = pltpu.make_async_remote_copy(
                src_ref=rs_ref.at[s],
                dst_ref=rs_ref.at[r],
                send_sem=rs_send_sems.at[s],
                recv_sem=rs_recv_sems.at[r],
                device_id=(right,),
                device_id_type=pl.DeviceIdType.MESH,
            )
            rdma.start()
            rdma.wait()
            if h < N_DEV - 2:
                c = (my - 2 - h) % N_DEV
                rs_ref[r, :, :] = rs_ref[r, :, :] + t_chunk(c)

        red = rs_ref[(N_DEV - 1) % 2, :, :] + t_chunk(my)
        res = jnp.dot(
            red, w_ref[:, :].astype(jnp.bfloat16),
            preferred_element_type=jnp.float32,
        )
        out_ref[pl.ds(my * m_chunk, m_chunk), :] = res

        ag_ref[0, :, :] = res.astype(jnp.bfloat16)
        for h in range(N_DEV - 1):
            s, r = h % 2, (h + 1) % 2
            rdma = pltpu.make_async_remote_copy(
                src_ref=ag_ref.at[s],
                dst_ref=ag_ref.at[r],
                send_sem=ag_send_sems.at[s],
                recv_sem=ag_recv_sems.at[r],
                device_id=(right,),
                device_id_type=pl.DeviceIdType.MESH,
            )
            rdma.start()
            rdma.wait()
            origin = (my - h - 1) % N_DEV
            out_ref[pl.ds(origin * m_chunk, m_chunk), :] = (
                ag_ref[r, :, :].astype(jnp.float32)
            )

    return pl.pallas_call(
        body,
        out_shape=jax.ShapeDtypeStruct((m_per, n), jnp.float32),
        in_specs=[
            pl.BlockSpec(memory_space=pltpu.VMEM),
            pl.BlockSpec(memory_space=pltpu.VMEM),
        ],
        out_specs=pl.BlockSpec(memory_space=pltpu.VMEM),
        scratch_shapes=[
            pltpu.VMEM((2, m_chunk, k), jnp.bfloat16),
            pltpu.VMEM((2, m_chunk, n), jnp.bfloat16),
            pltpu.SemaphoreType.DMA((2,)),
            pltpu.SemaphoreType.DMA((2,)),
            pltpu.SemaphoreType.DMA((2,)),
            pltpu.SemaphoreType.DMA((2,)),
        ],
        compiler_params=pltpu.CompilerParams(collective_id=0),
    )(t, W)


# device time: 98335 ns/iter; 1.6828x vs baseline; 1.6828x over previous
import jax
import jax.numpy as jnp
from jax import lax
from jax.experimental import pallas as pl
from jax.experimental.pallas import tpu as pltpu

N_DEV = 4


def kernel(t, W):
    m_per, k = t.shape
    _, n = W.shape
    m_chunk = m_per // N_DEV
    kh = k // 2
    nh = n // 2

    def body(t_ref, w_ref, out_ref,
             cw_ref, ccw_ref, agcw_ref, agccw_ref,
             cw_ssem, cw_rsem, ccw_ssem, ccw_rsem,
             agcw_ssem, agcw_rsem, agccw_ssem, agccw_rsem):
        my = lax.axis_index("i")
        left = (my - 1) % N_DEV
        right = (my + 1) % N_DEV

        barrier_sem = pltpu.get_barrier_semaphore()
        for nbr in (left, right):
            pl.semaphore_signal(
                barrier_sem, inc=1,
                device_id=(nbr,), device_id_type=pl.DeviceIdType.MESH,
            )
        pl.semaphore_wait(barrier_sem, 2)

        def t_sub(c, col0):
            return t_ref[
                pl.ds(c * m_chunk, m_chunk), pl.ds(col0, kh)
            ].astype(jnp.bfloat16)

        def hop(src, dst, ssem, rsem, slot_s, slot_r, target):
            return pltpu.make_async_remote_copy(
                src_ref=src.at[slot_s],
                dst_ref=dst.at[slot_r],
                send_sem=ssem.at[slot_s],
                recv_sem=rsem.at[slot_r],
                device_id=(target,),
                device_id_type=pl.DeviceIdType.MESH,
            )

        cw_ref[0, :, :] = t_sub((my - 1) % N_DEV, 0)
        ccw_ref[0, :, :] = t_sub((my + 1) % N_DEV, kh)
        for h in range(N_DEV - 1):
            s, r = h % 2, (h + 1) % 2
            rd_cw = hop(cw_ref, cw_ref, cw_ssem, cw_rsem, s, r, right)
            rd_ccw = hop(ccw_ref, ccw_ref, ccw_ssem, ccw_rsem, s, r, left)
            rd_cw.start()
            rd_ccw.start()
            rd_cw.wait()
            rd_ccw.wait()
            if h < N_DEV - 2:
                cw_ref[r, :, :] = cw_ref[r, :, :] + t_sub((my - 2 - h) % N_DEV, 0)
                ccw_ref[r, :, :] = ccw_ref[r, :, :] + t_sub((my + 2 + h) % N_DEV, kh)

        last = (N_DEV - 1) % 2
        red = jnp.concatenate(
            [cw_ref[last, :, :] + t_sub(my, 0),
             ccw_ref[last, :, :] + t_sub(my, kh)],
            axis=1,
        )
        res = jnp.dot(
            red, w_ref[:, :].astype(jnp.bfloat16),
            preferred_element_type=jnp.float32,
        )
        out_ref[pl.ds(my * m_chunk, m_chunk), :] = res

        agcw_ref[0, :, :] = res[:, :nh].astype(jnp.bfloat16)
        agccw_ref[0, :, :] = res[:, nh:].astype(jnp.bfloat16)
        for h in range(N_DEV - 1):
            s, r = h % 2, (h + 1) % 2
            rd_cw = hop(agcw_ref, agcw_ref, agcw_ssem, agcw_rsem, s, r, right)
            rd_ccw = hop(agccw_ref, agccw_ref, agccw_ssem, agccw_rsem, s, r, left)
            rd_cw.start()
            rd_ccw.start()
            rd_cw.wait()
            rd_ccw.wait()
            oc_cw = (my - 1 - h) % N_DEV
            oc_ccw = (my + 1 + h) % N_DEV
            out_ref[pl.ds(oc_cw * m_chunk, m_chunk), pl.ds(0, nh)] = (
                agcw_ref[r, :, :].astype(jnp.float32)
            )
            out_ref[pl.ds(oc_ccw * m_chunk, m_chunk), pl.ds(nh, nh)] = (
                agccw_ref[r, :, :].astype(jnp.float32)
            )

    return pl.pallas_call(
        body,
        out_shape=jax.ShapeDtypeStruct((m_per, n), jnp.float32),
        in_specs=[
            pl.BlockSpec(memory_space=pltpu.VMEM),
            pl.BlockSpec(memory_space=pltpu.VMEM),
        ],
        out_specs=pl.BlockSpec(memory_space=pltpu.VMEM),
        scratch_shapes=[
            pltpu.VMEM((2, m_chunk, kh), jnp.bfloat16),
            pltpu.VMEM((2, m_chunk, kh), jnp.bfloat16),
            pltpu.VMEM((2, m_chunk, nh), jnp.bfloat16),
            pltpu.VMEM((2, m_chunk, nh), jnp.bfloat16),
            pltpu.SemaphoreType.DMA((2,)),
            pltpu.SemaphoreType.DMA((2,)),
            pltpu.SemaphoreType.DMA((2,)),
            pltpu.SemaphoreType.DMA((2,)),
            pltpu.SemaphoreType.DMA((2,)),
            pltpu.SemaphoreType.DMA((2,)),
            pltpu.SemaphoreType.DMA((2,)),
            pltpu.SemaphoreType.DMA((2,)),
        ],
        compiler_params=pltpu.CompilerParams(collective_id=0),
    )(t, W)
